# baseline (speedup 1.0000x reference)
import functools

import jax
import jax.numpy as jnp
from jax.experimental import pallas as pl
from jax.experimental.pallas import tpu as pltpu

N_TOKENS = 4096
D_IN = 1024
D_OUT = 1024
RANK = 8
NUM_EXPERTS = 64
TOP_K = 2
LORA_ALPHA = 32.0
_SCALING = LORA_ALPHA / RANK

_BN = 512
_ER = NUM_EXPERTS * RANK

_DN_T = (((1,), (1,)), ((), ()))
_DN_N = (((1,), (0,)), ((), ()))


def _moe_lora_kernel(x_ref, wb_ref, wr_ref, a_ref, bf_ref, o_ref, w_s):
    i = pl.program_id(0)

    @pl.when(i == 0)
    def _init():
        w_s[:D_OUT, :] = wb_ref[:].astype(jnp.bfloat16)
        w_s[D_OUT:, :] = a_ref[:].astype(jnp.bfloat16)

    xb = x_ref[:]
    xb16 = xb.astype(jnp.bfloat16)
    logits = jax.lax.dot_general(
        xb, wr_ref[:], _DN_T, preferred_element_type=jnp.float32
    )
    idx1 = jnp.argmax(logits, axis=-1)
    m1 = jnp.max(logits, axis=-1)
    eiota = jax.lax.broadcasted_iota(jnp.int32, logits.shape, 1)
    masked = jnp.where(eiota == idx1[:, None], -jnp.inf, logits)
    idx2 = jnp.argmax(masked, axis=-1)
    m2 = jnp.max(masked, axis=-1)
    g2 = 1.0 / (1.0 + jnp.exp(m1 - m2))
    g1 = 1.0 - g2

    bh = jax.lax.dot_general(
        xb16, w_s[:], _DN_T, preferred_element_type=jnp.float32
    )
    base = bh[:, :D_OUT]
    h = bh[:, D_OUT:]
    gE = jnp.where(eiota == idx1[:, None], g1[:, None], 0.0) + jnp.where(
        eiota == idx2[:, None], g2[:, None], 0.0
    )
    ge = jnp.broadcast_to(gE[:, :, None], (gE.shape[0], NUM_EXPERTS, RANK)).reshape(
        gE.shape[0], _ER
    )
    hw = (h * ge).astype(jnp.bfloat16)
    lora = jax.lax.dot_general(
        hw, bf_ref[:], _DN_N, preferred_element_type=jnp.float32
    )
    o_ref[:] = base + lora


@functools.partial(jax.jit, static_argnames=())
def kernel(x, W_base, W_router, A, B):
    a2 = A.reshape(_ER, D_IN)
    bf16m = (
        (B * _SCALING).transpose(0, 2, 1).reshape(_ER, D_OUT).astype(jnp.bfloat16)
    )

    grid = (N_TOKENS // _BN,)
    return pl.pallas_call(
        _moe_lora_kernel,
        grid=grid,
        in_specs=[
            pl.BlockSpec((_BN, D_IN), lambda i: (i, 0)),
            pl.BlockSpec((D_OUT, D_IN), lambda i: (0, 0)),
            pl.BlockSpec((NUM_EXPERTS, D_IN), lambda i: (0, 0)),
            pl.BlockSpec((_ER, D_IN), lambda i: (0, 0)),
            pl.BlockSpec((_ER, D_OUT), lambda i: (0, 0)),
        ],
        out_specs=pl.BlockSpec((_BN, D_OUT), lambda i: (i, 0)),
        out_shape=jax.ShapeDtypeStruct((N_TOKENS, D_OUT), jnp.float32),
        scratch_shapes=[pltpu.VMEM((D_OUT + _ER, D_IN), jnp.bfloat16)],
    )(x, W_base, W_router, a2, bf16m)


# speedup vs baseline: 2.9531x; 2.9531x over previous
import functools

import jax
import jax.numpy as jnp
from jax.experimental import pallas as pl
from jax.experimental.pallas import tpu as pltpu

N_TOKENS = 4096
D_IN = 1024
D_OUT = 1024
RANK = 8
NUM_EXPERTS = 64
TOP_K = 2
LORA_ALPHA = 32.0
_SCALING = LORA_ALPHA / RANK

_BN = 1024
_ER = NUM_EXPERTS * RANK

_DN_T = (((1,), (1,)), ((), ()))
_DN_N = (((1,), (0,)), ((), ()))


def _moe_lora_kernel(x_ref, wb_ref, wr_ref, a_ref, bf_ref, o_ref, w_s):
    i = pl.program_id(0)

    @pl.when(i == 0)
    def _init():
        w_s[:D_OUT, :] = wb_ref[:].astype(jnp.bfloat16)
        w_s[D_OUT:, :] = a_ref[:].astype(jnp.bfloat16)

    xb = x_ref[:]
    xb16 = xb.astype(jnp.bfloat16)
    logits = jax.lax.dot_general(
        xb, wr_ref[:], _DN_T, preferred_element_type=jnp.float32
    )
    idx1 = jnp.argmax(logits, axis=-1)
    m1 = jnp.max(logits, axis=-1)
    eiota = jax.lax.broadcasted_iota(jnp.int32, logits.shape, 1)
    masked = jnp.where(eiota == idx1[:, None], -jnp.inf, logits)
    idx2 = jnp.argmax(masked, axis=-1)
    m2 = jnp.max(masked, axis=-1)
    g2 = 1.0 / (1.0 + jnp.exp(m1 - m2))
    g1 = 1.0 - g2

    bh = jax.lax.dot_general(
        xb16, w_s[:], _DN_T, preferred_element_type=jnp.float32
    )
    base = bh[:, :D_OUT]
    h = bh[:, D_OUT:]
    col_expert = jax.lax.broadcasted_iota(jnp.int32, h.shape, 1) // RANK
    ge = jnp.where(col_expert == idx1[:, None], g1[:, None], 0.0) + jnp.where(
        col_expert == idx2[:, None], g2[:, None], 0.0
    )
    hw = (h * ge).astype(jnp.bfloat16)
    lora = jax.lax.dot_general(
        hw, bf_ref[:], _DN_N, preferred_element_type=jnp.float32
    )
    o_ref[:] = base + lora


@functools.partial(jax.jit, static_argnames=())
def kernel(x, W_base, W_router, A, B):
    a2 = A.reshape(_ER, D_IN)
    bf16m = (
        (B * _SCALING).transpose(0, 2, 1).reshape(_ER, D_OUT).astype(jnp.bfloat16)
    )

    grid = (N_TOKENS // _BN,)
    return pl.pallas_call(
        _moe_lora_kernel,
        grid=grid,
        in_specs=[
            pl.BlockSpec((_BN, D_IN), lambda i: (i, 0)),
            pl.BlockSpec((D_OUT, D_IN), lambda i: (0, 0)),
            pl.BlockSpec((NUM_EXPERTS, D_IN), lambda i: (0, 0)),
            pl.BlockSpec((_ER, D_IN), lambda i: (0, 0)),
            pl.BlockSpec((_ER, D_OUT), lambda i: (0, 0)),
        ],
        out_specs=pl.BlockSpec((_BN, D_OUT), lambda i: (i, 0)),
        out_shape=jax.ShapeDtypeStruct((N_TOKENS, D_OUT), jnp.float32),
        scratch_shapes=[pltpu.VMEM((D_OUT + _ER, D_IN), jnp.bfloat16)],
    )(x, W_base, W_router, a2, bf16m)
